# Initial kernel scaffold; baseline (speedup 1.0000x reference)
import jax
import jax.numpy as jnp
from jax.experimental import pallas as pl


def kernel(h, features, up_edge_index, down_edge_index, up_W1, up_b1, up_W2, up_b2, down_W1, down_b1, down_W2, down_b2, gru_Wih, gru_Whh, gru_bih, gru_bhh):
    raise NotImplementedError("write your pallas kernel here")


# speedup vs baseline: 2.1707x; 2.1707x over previous
import functools

import jax
import jax.numpy as jnp
from jax import lax
from jax.experimental import pallas as pl
from jax.experimental.pallas import tpu as pltpu
from jax.experimental.pallas import tpu_sc as plsc

N = 10000
H = 256
HH = 128
E = 80000
CHUNK = 128

_DOT = functools.partial(
    jnp.dot, precision=lax.Precision.HIGHEST, preferred_element_type=jnp.float32
)


def _pre_body(h_ref, f_ref, w_ref, b_ref, wr_ref, *out_refs):
    x = _DOT(h_ref[...], w_ref[...]) + b_ref[...]
    x = x + f_ref[...] * wr_ref[...]
    for j in range(8):
        out_refs[j][...] = x[:, j * HH:(j + 1) * HH]


def _pre_projection(h, fcol2d, w_cat, b_cat, wr_cat):
    bn = 1000
    grid = N // bn
    return pl.pallas_call(
        _pre_body,
        grid=(grid,),
        in_specs=[
            pl.BlockSpec((bn, H), lambda i: (i, 0)),
            pl.BlockSpec((bn, 1), lambda i: (i, 0)),
            pl.BlockSpec((H, 4 * H), lambda i: (0, 0)),
            pl.BlockSpec((1, 4 * H), lambda i: (0, 0)),
            pl.BlockSpec((1, 4 * H), lambda i: (0, 0)),
        ],
        out_specs=[pl.BlockSpec((bn, HH), lambda i: (i, 0))] * 8,
        out_shape=[jax.ShapeDtypeStruct((N, HH), jnp.float32)] * 8,
    )(h, fcol2d, w_cat, b_cat, wr_cat)


def _sc_body(
    a_u0, a_u1, b_u0, b_u1, a_d0, a_d1, b_d0, b_d1,
    up_src, up_tgt, dn_src, dn_tgt,
    z2d,
    s_u0, s_u1, s_d0, s_d1, deg_part,
    idx_s, idx_t, a_buf, b_buf, deg_loc, s_sh, sem,
):
    c = lax.axis_index("c")
    s = lax.axis_index("s")

    planes = ((a_u0, a_u1), (b_u0, b_u1)), ((a_d0, a_d1), (b_d0, b_d1))
    s_outs = ((s_u0, s_u1), (s_d0, s_d1))
    srcs = (up_src, dn_src)
    tgts = (up_tgt, dn_tgt)

    zeros16 = jnp.zeros((16,), jnp.float32)
    ones16 = jnp.ones((16,), jnp.float32)

    nchunks_total = E // CHUNK
    q, r = divmod(nchunks_total, 16)
    my_chunks = q + jnp.where(s < r, 1, 0)

    for t in range(2):
        for ci in range(2):
            @pl.when(c == ci)
            def _(t=t, ci=ci):
                pltpu.sync_copy(z2d, s_sh.at[pl.ds(s * 624, 624)])

                @pl.when(s == 15)
                def _():
                    pltpu.sync_copy(
                        z2d.at[pl.ds(0, 16)], s_sh.at[pl.ds(9984, 16)]
                    )
                if ci == 0:
                    def zero_body(i, carry):
                        deg_loc[pl.ds(i * 16, 16)] = zeros16
                        return carry

                    lax.fori_loop(0, N // 16 + 1, zero_body, 0, unroll=False)
                plsc.subcore_barrier()

                def chunk_body(j, carry):
                    base = (s + j * 16) * CHUNK
                    base = pl.multiple_of(base, CHUNK)
                    pltpu.sync_copy(srcs[t].at[pl.ds(base, CHUNK)], idx_s)
                    pltpu.sync_copy(tgts[t].at[pl.ds(base, CHUNK)], idx_t)
                    cp3 = pltpu.async_copy(planes[t][0][ci].at[idx_s], a_buf, sem)
                    cp4 = pltpu.async_copy(planes[t][1][ci].at[idx_t], b_buf, sem)
                    cp3.wait()
                    cp4.wait()

                    def edge_body(e, carry2):
                        for i in range(8):
                            sl = pl.ds(i * 16, 16)
                            p = a_buf[e, sl] + b_buf[e, sl]
                            sg = 1.0 / (1.0 + jnp.exp(-p))
                            a_buf[e, sl] = p * sg
                        return carry2

                    lax.fori_loop(0, CHUNK, edge_body, 0, unroll=False)
                    pltpu.sync_copy(a_buf, s_sh.at[idx_t], add=True)
                    if ci == 0:
                        for i in range(8):
                            iv = idx_t[pl.ds(i * 16, 16)]
                            plsc.addupdate_scatter(deg_loc, [iv], ones16)
                    return carry

                lax.fori_loop(0, my_chunks, chunk_body, 0, unroll=False)
                plsc.subcore_barrier()

                pltpu.sync_copy(
                    s_sh.at[pl.ds(s * 624, 624)],
                    s_outs[t][ci].at[pl.ds(s * 624, 624)],
                )

                @pl.when(s == 15)
                def _(t=t, ci=ci):
                    pltpu.sync_copy(
                        s_sh.at[pl.ds(9984, 16)],
                        s_outs[t][ci].at[pl.ds(9984, 16)],
                    )
                if ci == 0:
                    pltpu.sync_copy(
                        deg_loc.at[pl.ds(0, N)],
                        deg_part.at[pl.ds((t * 16 + s) * 10240, N)],
                    )
                plsc.subcore_barrier()


def _sc_edges(planes8, up_src, up_tgt, dn_src, dn_tgt):
    z2d = jnp.zeros((624, HH), jnp.float32)
    mesh = plsc.VectorSubcoreMesh(
        core_axis_name="c", subcore_axis_name="s", num_cores=2, num_subcores=16
    )
    fn = pl.kernel(
        _sc_body,
        out_type=(
            [jax.ShapeDtypeStruct((N, HH), jnp.float32)] * 4
            + [jax.ShapeDtypeStruct((2 * 16 * 10240,), jnp.float32)]
        ),
        mesh=mesh,
        compiler_params=pltpu.CompilerParams(needs_layout_passes=False),
        scratch_types=[
            pltpu.VMEM((CHUNK,), jnp.int32),
            pltpu.VMEM((CHUNK,), jnp.int32),
            pltpu.VMEM((CHUNK, HH), jnp.float32),
            pltpu.VMEM((CHUNK, HH), jnp.float32),
            pltpu.VMEM((N + 16,), jnp.float32),
            pltpu.VMEM_SHARED((N, HH), jnp.float32),
            pltpu.SemaphoreType.DMA,
        ],
    )
    return fn(
        planes8[0], planes8[1], planes8[2], planes8[3],
        planes8[4], planes8[5], planes8[6], planes8[7],
        up_src, up_tgt, dn_src, dn_tgt,
        z2d,
    )


def _post_body(su0, su1, sd0, sd1, bias2, h_ref,
               w2ua, w2ub, w2da, w2db, wih, whh, bih, bhh, out_ref):
    tot = (
        _DOT(su0[...], w2ua[...]) + _DOT(su1[...], w2ub[...])
        + _DOT(sd0[...], w2da[...]) + _DOT(sd1[...], w2db[...])
        + bias2[...]
    )
    gx = _DOT(tot, wih[...]) + bih[...]
    h = h_ref[...]
    gh = _DOT(h, whh[...]) + bhh[...]
    r = jax.nn.sigmoid(gx[:, :H] + gh[:, :H])
    z = jax.nn.sigmoid(gx[:, H:2 * H] + gh[:, H:2 * H])
    n = jnp.tanh(gx[:, 2 * H:] + r * gh[:, 2 * H:])
    out_ref[...] = (1.0 - z) * n + z * h


def _post_gru(su0, su1, sd0, sd1, bias2, h, w2ua, w2ub, w2da, w2db,
              wih, whh, bih, bhh):
    bn = 1000
    grid = N // bn
    full = lambda shp: pl.BlockSpec(shp, lambda i: tuple(0 for _ in shp))
    row = lambda shp: pl.BlockSpec(shp, lambda i: (i, 0))
    return pl.pallas_call(
        _post_body,
        grid=(grid,),
        in_specs=[
            row((bn, HH)), row((bn, HH)), row((bn, HH)), row((bn, HH)),
            row((bn, H)), row((bn, H)),
            full((HH, H)), full((HH, H)), full((HH, H)), full((HH, H)),
            full((H, 3 * H)), full((H, 3 * H)),
            full((1, 3 * H)), full((1, 3 * H)),
        ],
        out_specs=row((bn, H)),
        out_shape=jax.ShapeDtypeStruct((N, H), jnp.float32),
    )(su0, su1, sd0, sd1, bias2, h, w2ua, w2ub, w2da, w2db, wih, whh, bih, bhh)


@jax.jit
def _impl(h, features, up_edge_index, down_edge_index,
          up_W1, up_b1, up_W2, up_b2,
          down_W1, down_b1, down_W2, down_b2,
          gru_Wih, gru_Whh, gru_bih, gru_bhh):
    w_cat = jnp.concatenate(
        [up_W1[:H], up_W1[H:2 * H], down_W1[:H], down_W1[H:2 * H]], axis=1
    )
    b_cat = jnp.concatenate(
        [up_b1, jnp.zeros((H,), jnp.float32),
         down_b1, jnp.zeros((H,), jnp.float32)]
    ).reshape(1, 4 * H)
    wr_cat = jnp.concatenate(
        [up_W1[2 * H], -up_W1[2 * H], down_W1[2 * H], -down_W1[2 * H]]
    ).reshape(1, 4 * H)
    fcol2d = features[:, 0:1]

    planes8 = _pre_projection(h, fcol2d, w_cat, b_cat, wr_cat)

    s_u0, s_u1, s_d0, s_d1, deg_part = _sc_edges(
        planes8,
        up_edge_index[0], up_edge_index[1],
        down_edge_index[0], down_edge_index[1],
    )

    deg = deg_part.reshape(2, 16, 10240)[:, :, :N].sum(axis=1)
    bias2 = deg[0][:, None] * up_b2[None, :] + deg[1][:, None] * down_b2[None, :]

    return _post_gru(
        s_u0, s_u1, s_d0, s_d1, bias2, h,
        up_W2[:HH], up_W2[HH:], down_W2[:HH], down_W2[HH:],
        gru_Wih, gru_Whh,
        gru_bih.reshape(1, 3 * H), gru_bhh.reshape(1, 3 * H),
    )


def kernel(h, features, up_edge_index, down_edge_index,
           up_W1, up_b1, up_W2, up_b2,
           down_W1, down_b1, down_W2, down_b2,
           gru_Wih, gru_Whh, gru_bih, gru_bhh):
    return _impl(h, features, up_edge_index, down_edge_index,
                 up_W1, up_b1, up_W2, up_b2,
                 down_W1, down_b1, down_W2, down_b2,
                 gru_Wih, gru_Whh, gru_bih, gru_bhh)


# speedup vs baseline: 3.9166x; 1.8043x over previous
import functools

import jax
import jax.numpy as jnp
from jax import lax
from jax.experimental import pallas as pl
from jax.experimental.pallas import tpu as pltpu
from jax.experimental.pallas import tpu_sc as plsc

N = 10000
H = 256
HH = 128
E = 80000
CHUNK = 40
NSLOT = 2
PER_SUB = E // 16 // CHUNK

_DOT = functools.partial(
    jnp.dot, precision=lax.Precision.DEFAULT, preferred_element_type=jnp.float32
)


def _pre_body(h_ref, f_ref, w_ref, b_ref, wr_ref, *out_refs):
    x = _DOT(h_ref[...], w_ref[...]) + b_ref[...]
    x = x + f_ref[...] * wr_ref[...]
    for j in range(8):
        out_refs[j][...] = x[:, j * HH:(j + 1) * HH]


def _pre_projection(h, fcol2d, w_cat, b_cat, wr_cat):
    bn = 1000
    grid = N // bn
    return pl.pallas_call(
        _pre_body,
        grid=(grid,),
        in_specs=[
            pl.BlockSpec((bn, H), lambda i: (i, 0)),
            pl.BlockSpec((bn, 1), lambda i: (i, 0)),
            pl.BlockSpec((H, 4 * H), lambda i: (0, 0)),
            pl.BlockSpec((1, 4 * H), lambda i: (0, 0)),
            pl.BlockSpec((1, 4 * H), lambda i: (0, 0)),
        ],
        out_specs=[pl.BlockSpec((bn, HH), lambda i: (i, 0))] * 8,
        out_shape=[jax.ShapeDtypeStruct((N, HH), jnp.float32)] * 8,
    )(h, fcol2d, w_cat, b_cat, wr_cat)


def _sc_body(
    a_u0, a_u1, b_u0, b_u1, a_d0, a_d1, b_d0, b_d1,
    up_src, up_tgt, dn_src, dn_tgt,
    z2d,
    s_u0, s_u1, s_d0, s_d1, deg_part,
    idx_s, idx_t, a_buf, b_buf, deg_loc, s_sh,
    gs0, gs1, ss0, ss1,
):
    c = lax.axis_index("c")
    s = lax.axis_index("s")

    planes = ((a_u0, a_u1), (b_u0, b_u1)), ((a_d0, a_d1), (b_d0, b_d1))
    s_outs = ((s_u0, s_u1), (s_d0, s_d1))
    srcs = (up_src, dn_src)
    tgts = (up_tgt, dn_tgt)
    gsem = (gs0, gs1)
    ssem = (ss0, ss1)

    zeros16 = jnp.zeros((16,), jnp.float32)
    ones16 = jnp.ones((16,), jnp.float32)
    lane = lax.iota(jnp.int32, 16)

    for t in range(2):
        for ci in range(2):
            @pl.when(c == ci)
            def _(t=t, ci=ci):
                aplane = planes[t][0][ci]
                bplane = planes[t][1][ci]

                pltpu.sync_copy(z2d, s_sh.at[pl.ds(s * 624, 624)])

                @pl.when(s == 15)
                def _():
                    pltpu.sync_copy(
                        z2d.at[pl.ds(0, 16)], s_sh.at[pl.ds(9984, 16)]
                    )
                if ci == 0:
                    def zero_body(i, carry):
                        deg_loc[pl.ds(i * 16, 16)] = zeros16
                        return carry

                    lax.fori_loop(0, N // 16, zero_body, 0, unroll=False)
                plsc.subcore_barrier()

                base = s * (E // 16)
                base = pl.multiple_of(base, 8)
                pltpu.sync_copy(srcs[t].at[pl.ds(base, E // 16)], idx_s)
                pltpu.sync_copy(tgts[t].at[pl.ds(base, E // 16)], idx_t)

                def wait_scatter(slot):
                    pltpu.make_async_copy(
                        a_buf.at[pl.ds(slot * CHUNK, CHUNK)],
                        s_sh.at[idx_t.at[pl.ds(0, CHUNK)]],
                        ssem[slot],
                    ).wait()

                def stage_a(q, slot, guard_wait):
                    if guard_wait:
                        @pl.when(q >= NSLOT)
                        def _():
                            wait_scatter(slot)
                    sl = pl.ds(slot * CHUNK, CHUNK)
                    off = q * CHUNK
                    pltpu.async_copy(
                        aplane.at[idx_s.at[pl.ds(off, CHUNK)]],
                        a_buf.at[sl], gsem[slot],
                    )
                    pltpu.async_copy(
                        bplane.at[idx_t.at[pl.ds(off, CHUNK)]],
                        b_buf.at[sl], gsem[slot],
                    )

                def stage_b(q, slot):
                    sl = pl.ds(slot * CHUNK, CHUNK)
                    off = q * CHUNK
                    pltpu.make_async_copy(
                        aplane.at[idx_s.at[pl.ds(off, CHUNK)]],
                        a_buf.at[sl], gsem[slot],
                    ).wait()
                    pltpu.make_async_copy(
                        bplane.at[idx_t.at[pl.ds(off, CHUNK)]],
                        b_buf.at[sl], gsem[slot],
                    ).wait()

                    def edge_body(e, carry2):
                        row = slot * CHUNK + e
                        for i in range(8):
                            g = pl.ds(i * 16, 16)
                            p = a_buf[row, g] + b_buf[row, g]
                            a_buf[row, g] = p / (1.0 + jnp.exp(-p))
                        return carry2

                    lax.fori_loop(0, CHUNK, edge_body, 0, unroll=False)
                    off = q * CHUNK
                    if ci == 0:
                        plsc.addupdate_scatter(
                            deg_loc, [idx_t[pl.ds(off, 16)]], ones16
                        )
                        plsc.addupdate_scatter(
                            deg_loc, [idx_t[pl.ds(off + 16, 16)]], ones16
                        )
                        plsc.addupdate_scatter(
                            deg_loc, [idx_t[pl.ds(off + 24, 16)]], ones16,
                            mask=lane >= 8,
                        )
                    pltpu.async_copy(
                        a_buf.at[sl],
                        s_sh.at[idx_t.at[pl.ds(off, CHUNK)]],
                        ssem[slot], add=True,
                    )

                stage_a(0, 0, False)

                def pipe_body(i, carry):
                    q0 = i * 2
                    stage_a(q0 + 1, 1, True)
                    stage_b(q0, 0)
                    stage_a(q0 + 2, 0, True)
                    stage_b(q0 + 1, 1)
                    return carry

                lax.fori_loop(0, (PER_SUB - 1) // 2, pipe_body, 0,
                              unroll=False)
                stage_b(PER_SUB - 1, (PER_SUB - 1) % 2)
                for slot in range(NSLOT):
                    wait_scatter(slot)
                plsc.subcore_barrier()

                pltpu.sync_copy(
                    s_sh.at[pl.ds(s * 624, 624)],
                    s_outs[t][ci].at[pl.ds(s * 624, 624)],
                )

                @pl.when(s == 15)
                def _(t=t, ci=ci):
                    pltpu.sync_copy(
                        s_sh.at[pl.ds(9984, 16)],
                        s_outs[t][ci].at[pl.ds(9984, 16)],
                    )
                if ci == 0:
                    pltpu.sync_copy(
                        deg_loc.at[pl.ds(0, N)],
                        deg_part.at[pl.ds((t * 16 + s) * 10240, N)],
                    )
                plsc.subcore_barrier()


def _sc_edges(planes8, up_src, up_tgt, dn_src, dn_tgt):
    z2d = jnp.zeros((624, HH), jnp.float32)
    mesh = plsc.VectorSubcoreMesh(
        core_axis_name="c", subcore_axis_name="s", num_cores=2, num_subcores=16
    )
    fn = pl.kernel(
        _sc_body,
        out_type=(
            [jax.ShapeDtypeStruct((N, HH), jnp.float32)] * 4
            + [jax.ShapeDtypeStruct((2 * 16 * 10240,), jnp.float32)]
        ),
        mesh=mesh,
        compiler_params=pltpu.CompilerParams(needs_layout_passes=False),
        scratch_types=[
            pltpu.VMEM((E // 16,), jnp.int32),
            pltpu.VMEM((E // 16,), jnp.int32),
            pltpu.VMEM((NSLOT * CHUNK, HH), jnp.float32),
            pltpu.VMEM((NSLOT * CHUNK, HH), jnp.float32),
            pltpu.VMEM((N,), jnp.float32),
            pltpu.VMEM_SHARED((N, HH), jnp.float32),
            pltpu.SemaphoreType.DMA,
            pltpu.SemaphoreType.DMA,
            pltpu.SemaphoreType.DMA,
            pltpu.SemaphoreType.DMA,
        ],
    )
    return fn(
        planes8[0], planes8[1], planes8[2], planes8[3],
        planes8[4], planes8[5], planes8[6], planes8[7],
        up_src, up_tgt, dn_src, dn_tgt,
        z2d,
    )


def _post_body(su0, su1, sd0, sd1, dpart, b2u, b2d, h_ref,
               w2ua, w2ub, w2da, w2db, wih, whh, bih, bhh, out_ref):
    dp = dpart[...]
    degu = jnp.sum(dp[0], axis=1)
    degd = jnp.sum(dp[1], axis=1)
    tot = (
        _DOT(su0[...], w2ua[...]) + _DOT(su1[...], w2ub[...])
        + _DOT(sd0[...], w2da[...]) + _DOT(sd1[...], w2db[...])
        + degu[:, None] * b2u[...] + degd[:, None] * b2d[...]
    )
    gx = _DOT(tot, wih[...]) + bih[...]
    h = h_ref[...]
    gh = _DOT(h, whh[...]) + bhh[...]
    r = jax.nn.sigmoid(gx[:, :H] + gh[:, :H])
    z = jax.nn.sigmoid(gx[:, H:2 * H] + gh[:, H:2 * H])
    n = jnp.tanh(gx[:, 2 * H:] + r * gh[:, 2 * H:])
    out_ref[...] = (1.0 - z) * n + z * h


def _post_gru(su0, su1, sd0, sd1, dpart, b2u, b2d, h, w2ua, w2ub, w2da, w2db,
              wih, whh, bih, bhh):
    bn = 1000
    grid = N // bn
    full = lambda shp: pl.BlockSpec(shp, lambda i: tuple(0 for _ in shp))
    row = lambda shp: pl.BlockSpec(shp, lambda i: (i, 0))
    return pl.pallas_call(
        _post_body,
        grid=(grid,),
        in_specs=[
            row((bn, HH)), row((bn, HH)), row((bn, HH)), row((bn, HH)),
            pl.BlockSpec((2, bn, 16), lambda i: (0, i, 0)),
            full((1, H)), full((1, H)),
            row((bn, H)),
            full((HH, H)), full((HH, H)), full((HH, H)), full((HH, H)),
            full((H, 3 * H)), full((H, 3 * H)),
            full((1, 3 * H)), full((1, 3 * H)),
        ],
        out_specs=row((bn, H)),
        out_shape=jax.ShapeDtypeStruct((N, H), jnp.float32),
    )(su0, su1, sd0, sd1, dpart, b2u, b2d, h,
      w2ua, w2ub, w2da, w2db, wih, whh, bih, bhh)


@jax.jit
def _impl(h, features, up_edge_index, down_edge_index,
          up_W1, up_b1, up_W2, up_b2,
          down_W1, down_b1, down_W2, down_b2,
          gru_Wih, gru_Whh, gru_bih, gru_bhh):
    w_cat = jnp.concatenate(
        [up_W1[:H], up_W1[H:2 * H], down_W1[:H], down_W1[H:2 * H]], axis=1
    )
    b_cat = jnp.concatenate(
        [up_b1, jnp.zeros((H,), jnp.float32),
         down_b1, jnp.zeros((H,), jnp.float32)]
    ).reshape(1, 4 * H)
    wr_cat = jnp.concatenate(
        [up_W1[2 * H], -up_W1[2 * H], down_W1[2 * H], -down_W1[2 * H]]
    ).reshape(1, 4 * H)
    fcol2d = features[:, 0:1]

    planes8 = _pre_projection(h, fcol2d, w_cat, b_cat, wr_cat)

    s_u0, s_u1, s_d0, s_d1, deg_part = _sc_edges(
        planes8,
        up_edge_index[0], up_edge_index[1],
        down_edge_index[0], down_edge_index[1],
    )

    dpart = deg_part.reshape(2, 16, 10240).transpose(0, 2, 1)[:, :N, :]

    return _post_gru(
        s_u0, s_u1, s_d0, s_d1, dpart,
        up_b2.reshape(1, H), down_b2.reshape(1, H), h,
        up_W2[:HH], up_W2[HH:], down_W2[:HH], down_W2[HH:],
        gru_Wih, gru_Whh,
        gru_bih.reshape(1, 3 * H), gru_bhh.reshape(1, 3 * H),
    )


def kernel(h, features, up_edge_index, down_edge_index,
           up_W1, up_b1, up_W2, up_b2,
           down_W1, down_b1, down_W2, down_b2,
           gru_Wih, gru_Whh, gru_bih, gru_bhh):
    return _impl(h, features, up_edge_index, down_edge_index,
                 up_W1, up_b1, up_W2, up_b2,
                 down_W1, down_b1, down_W2, down_b2,
                 gru_Wih, gru_Whh, gru_bih, gru_bhh)
